# baseline (speedup 1.0000x reference)
import functools

import jax
import jax.numpy as jnp
from jax import lax
from jax.experimental import pallas as pl
from jax.experimental.pallas import tpu as pltpu
from jax.experimental.pallas import tpu_sc as plsc

N_VOCAB = 1000000
N_EMBED = 64
N_TOKEN = 50
BATCH = 4096

NC, NS = 2, 16
NW = NC * NS
BPW = BATCH // NW
LANES = 16
KQ = BPW // LANES

_GB = 2048
_NPAIR_BLOCKS = (N_VOCAB + _GB - 1) // _GB
_NPAIR = _NPAIR_BLOCKS * (_GB // 2)


def _pack_body(tp_ref, eye_ref, o_ref):
    x = tp_ref[...]
    y = lax.dot_general(x, eye_ref[...],
                        (((0,), (0,)), ((), ())),
                        preferred_element_type=jnp.float32)
    o_ref[...] = jnp.concatenate([y[: _GB // 2], y[_GB // 2:]], axis=1)


def _pack_table(table_t):
    eye = jnp.eye(N_EMBED, dtype=jnp.float32)
    return pl.pallas_call(
        _pack_body,
        grid=(_NPAIR_BLOCKS,),
        in_specs=[
            pl.BlockSpec((N_EMBED, _GB), lambda j: (0, j)),
            pl.BlockSpec((N_EMBED, N_EMBED), lambda j: (0, 0)),
        ],
        out_specs=pl.BlockSpec((_GB // 2, 128), lambda j: (j, 0)),
        out_shape=jax.ShapeDtypeStruct((_NPAIR, 128), jnp.float32),
        compiler_params=pltpu.CompilerParams(fuse_transposed_lhs_in_matmul=True),
    )(table_t, eye)


def _sc_body(tok_hbm, tab_hbm, pos_hbm, out_hbm,
             idx_v, pidx_v, pos_v, buf0, buf1, ob0, ob1,
             gsem0, gsem1, osem0, osem1):
    wid = lax.axis_index("s") * NC + lax.axis_index("c")
    b0 = wid * BPW

    pltpu.sync_copy(tok_hbm.at[:, pl.ds(b0, BPW)], idx_v)
    pltpu.sync_copy(pos_hbm, pos_v)

    def prep_row(t, c):
        for j in range(KQ):
            sl = pl.ds(j * LANES, LANES)
            v = idx_v[t, sl]
            pidx_v[t, sl] = lax.bitwise_or(
                lax.shift_left(lax.shift_right_logical(v, 11), 10),
                lax.bitwise_and(v, 1023))
        return c

    lax.fori_loop(0, N_TOKEN, prep_row, 0)

    iota16 = lax.iota(jnp.int32, LANES)
    rows = [iota16 + k * LANES for k in range(KQ)]
    bufs = (buf0, buf1)
    obs = (ob0, ob1)
    gsems = (gsem0, gsem1)
    osems = (osem0, osem1)
    ghandles = [None, None]
    ohandles = [None, None]

    def start_gather(t):
        b = t % 2
        ghandles[b] = pltpu.async_copy(
            tab_hbm.at[pidx_v.at[t]], bufs[b], gsems[b])

    def compute(t, buf, ob):
        tfull = jnp.full((LANES,), t, jnp.int32)
        hvs = []
        for k in range(KQ):
            v = idx_v[t, pl.ds(k * LANES, LANES)]
            hvs.append(lax.shift_right_logical(
                lax.bitwise_and(v, 1024), 4))

        def e_body(e, c):
            pv = plsc.load_gather(
                pos_v, [tfull, jnp.full((LANES,), e, jnp.int32)])
            for k in range(KQ):
                val = plsc.load_gather(buf, [rows[k], hvs[k] + e])
                ob[e, pl.ds(k * LANES, LANES)] = val + pv
            return c

        lax.fori_loop(0, N_EMBED, e_body, 0)

    start_gather(0)
    start_gather(1)
    for t in range(N_TOKEN):
        b = t % 2
        ghandles[b].wait()
        if ohandles[b] is not None:
            ohandles[b].wait()
            ohandles[b] = None
        compute(t, bufs[b], obs[b])
        ohandles[b] = pltpu.async_copy(
            obs[b], out_hbm.at[t, :, pl.ds(b0, BPW)], osems[b])
        if t + 2 < N_TOKEN:
            start_gather(t + 2)
    for b in range(2):
        if ohandles[b] is not None:
            ohandles[b].wait()


@jax.jit
def _run(tokens_t, table_t, pos):
    table_pairs = _pack_table(table_t)
    sc = functools.partial(
        pl.kernel,
        mesh=plsc.VectorSubcoreMesh(core_axis_name="c", subcore_axis_name="s"),
        compiler_params=pltpu.CompilerParams(
            use_tc_tiling_on_sc=True, needs_layout_passes=False),
        out_type=jax.ShapeDtypeStruct((N_TOKEN, N_EMBED, BATCH), jnp.float32),
        scratch_types=[
            pltpu.VMEM((N_TOKEN, BPW), jnp.int32),
            pltpu.VMEM((N_TOKEN, BPW), jnp.int32),
            pltpu.VMEM((N_TOKEN, N_EMBED), jnp.float32),
            pltpu.VMEM((BPW, 128), jnp.float32),
            pltpu.VMEM((BPW, 128), jnp.float32),
            pltpu.VMEM((N_EMBED, BPW), jnp.float32),
            pltpu.VMEM((N_EMBED, BPW), jnp.float32),
            pltpu.SemaphoreType.DMA,
            pltpu.SemaphoreType.DMA,
            pltpu.SemaphoreType.DMA,
            pltpu.SemaphoreType.DMA,
        ],
    )
    return sc(_sc_body)(tokens_t, table_pairs, pos)


def kernel(tokens, token_embedding, position_embedding):
    tokens_t = jnp.asarray(tokens, jnp.int32).T
    table_t = token_embedding.T
    out_t = _run(tokens_t, table_t, position_embedding)
    return jnp.transpose(out_t, (2, 0, 1))
